# Initial kernel scaffold; baseline (speedup 1.0000x reference)
import jax
import jax.numpy as jnp
from jax.experimental import pallas as pl


def kernel(z_q, edge_index, Wi, bi, Wc1, bc1, Wc2, bc2, gamma, beta, Wo1, bo1, Wo2, bo2):
    raise NotImplementedError("write your pallas kernel here")


# speedup vs baseline: 2.9186x; 2.9186x over previous
import functools

import jax
import jax.numpy as jnp
from jax import lax
from jax.experimental import pallas as pl
from jax.experimental.pallas import tpu as pltpu
from jax.experimental.pallas import tpu_sc as plsc

HID = 128
N_NODES = 10000
N_PAD = 10240
E_EDGES = 320000
GROUP = 128
N_TILES = 32
G_TOTAL = 2560
G_PER_TILE = G_TOTAL // N_TILES
E_PAD = G_TOTAL * GROUP
ROWS_PER_TILE = N_PAD // 16
DUMMY_DST = N_NODES + 16

TC_BLOCK = 512
TC_GRID = N_PAD // TC_BLOCK


def _sc_agg_body(h_hbm, srcg_hbm, dstg_hbm, out_hbm,
                 src_v, dst_v, rows_v, agg_sh, sem):
    c = lax.axis_index("c")
    s = lax.axis_index("s")
    tile = c * 16 + s

    pltpu.sync_copy(srcg_hbm.at[pl.ds(tile * G_PER_TILE, G_PER_TILE)], src_v)
    pltpu.sync_copy(dstg_hbm.at[pl.ds(tile * G_PER_TILE, G_PER_TILE)], dst_v)

    def zrow(i, carry):
        for j in range(8):
            rows_v[i, pl.ds(j * 16, 16)] = jnp.zeros((16,), jnp.float32)
        return carry
    lax.fori_loop(0, GROUP, zrow, 0)
    for z in range(ROWS_PER_TILE // GROUP):
        pltpu.sync_copy(rows_v, agg_sh.at[pl.ds(s * ROWS_PER_TILE + z * GROUP, GROUP)])
    plsc.subcore_barrier()

    def body(g, carry):
        pltpu.async_copy(h_hbm.at[src_v.at[g]], rows_v, sem).wait()
        pltpu.sync_copy(rows_v, agg_sh.at[dst_v.at[g]], add=True)
        return carry
    lax.fori_loop(0, G_PER_TILE, body, 0)
    plsc.subcore_barrier()

    for z in range(ROWS_PER_TILE // GROUP):
        r0 = s * ROWS_PER_TILE + z * GROUP
        pltpu.sync_copy(agg_sh.at[pl.ds(r0, GROUP)], rows_v)
        pltpu.sync_copy(rows_v, out_hbm.at[c, pl.ds(r0, GROUP)])


_sc_agg = pl.kernel(
    _sc_agg_body,
    out_type=jax.ShapeDtypeStruct((2, N_PAD, HID), jnp.float32),
    mesh=plsc.VectorSubcoreMesh(core_axis_name="c", subcore_axis_name="s"),
    scratch_types=[
        pltpu.VMEM((G_PER_TILE, GROUP), jnp.int32),
        pltpu.VMEM((G_PER_TILE, GROUP), jnp.int32),
        pltpu.VMEM((GROUP, HID), jnp.float32),
        pltpu.VMEM_SHARED((N_PAD, HID), jnp.float32),
        pltpu.SemaphoreType.DMA,
    ],
)


def _proj_body(z_ref, wi_ref, bi_ref, o_ref):
    o_ref[...] = (
        jnp.dot(z_ref[...], wi_ref[...], preferred_element_type=jnp.float32)
        + bi_ref[...]
    )


def _layer_body(h_ref, a_ref, w1_ref, b1_ref, w2_ref, b2_ref, g_ref, be_ref,
                o_ref):
    h = h_ref[...]
    x = h + a_ref[0] + a_ref[1]
    t = jnp.maximum(
        jnp.dot(x, w1_ref[...], preferred_element_type=jnp.float32) + b1_ref[...],
        0.0)
    t = jnp.dot(t, w2_ref[...], preferred_element_type=jnp.float32) + b2_ref[...]
    mu = jnp.mean(t, axis=-1, keepdims=True)
    var = jnp.mean((t - mu) ** 2, axis=-1, keepdims=True)
    t = (t - mu) / jnp.sqrt(var + 1e-5) * g_ref[...] + be_ref[...]
    o_ref[...] = h + jnp.maximum(t, 0.0)


def _layer_out_body(h_ref, a_ref, w1_ref, b1_ref, w2_ref, b2_ref, g_ref,
                    be_ref, wo1_ref, bo1_ref, wo2_ref, bo2_ref, o_ref):
    h = h_ref[...]
    x = h + a_ref[0] + a_ref[1]
    t = jnp.maximum(
        jnp.dot(x, w1_ref[...], preferred_element_type=jnp.float32) + b1_ref[...],
        0.0)
    t = jnp.dot(t, w2_ref[...], preferred_element_type=jnp.float32) + b2_ref[...]
    mu = jnp.mean(t, axis=-1, keepdims=True)
    var = jnp.mean((t - mu) ** 2, axis=-1, keepdims=True)
    t = (t - mu) / jnp.sqrt(var + 1e-5) * g_ref[...] + be_ref[...]
    h = h + jnp.maximum(t, 0.0)
    u = jnp.maximum(
        jnp.dot(h, wo1_ref[...], preferred_element_type=jnp.float32)
        + bo1_ref[...], 0.0)
    o_ref[...] = (
        jnp.dot(u, wo2_ref[...], preferred_element_type=jnp.float32)
        + bo2_ref[...])


def _row_spec():
    return pl.BlockSpec((TC_BLOCK, HID), lambda i: (i, 0))


def _agg_spec():
    return pl.BlockSpec((2, TC_BLOCK, HID), lambda i: (0, i, 0))


def _w_spec():
    return pl.BlockSpec((HID, HID), lambda i: (0, 0))


def _b_spec():
    return pl.BlockSpec((1, HID), lambda i: (0, 0))


_proj = pl.pallas_call(
    _proj_body,
    grid=(TC_GRID,),
    in_specs=[_row_spec(), _w_spec(), _b_spec()],
    out_specs=_row_spec(),
    out_shape=jax.ShapeDtypeStruct((N_PAD, HID), jnp.float32),
)

_layer = pl.pallas_call(
    _layer_body,
    grid=(TC_GRID,),
    in_specs=[_row_spec(), _agg_spec(), _w_spec(), _b_spec(), _w_spec(),
              _b_spec(), _b_spec(), _b_spec()],
    out_specs=_row_spec(),
    out_shape=jax.ShapeDtypeStruct((N_PAD, HID), jnp.float32),
)

_layer_out = pl.pallas_call(
    _layer_out_body,
    grid=(TC_GRID,),
    in_specs=[_row_spec(), _agg_spec(), _w_spec(), _b_spec(), _w_spec(),
              _b_spec(), _b_spec(), _b_spec(), _w_spec(), _b_spec(),
              _w_spec(), _b_spec()],
    out_specs=_row_spec(),
    out_shape=jax.ShapeDtypeStruct((N_PAD, HID), jnp.float32),
)


def kernel(z_q, edge_index, Wi, bi, Wc1, bc1, Wc2, bc2, gamma, beta,
           Wo1, bo1, Wo2, bo2):
    Bb, Nn, code = z_q.shape
    L = Wc1.shape[0]

    ei = edge_index.reshape(2, -1).astype(jnp.int32)
    src = jnp.concatenate(
        [ei[0], jnp.zeros((E_PAD - E_EDGES,), jnp.int32)]).reshape(G_TOTAL, GROUP)
    dst = jnp.concatenate(
        [ei[1], jnp.full((E_PAD - E_EDGES,), DUMMY_DST, jnp.int32)]
    ).reshape(G_TOTAL, GROUP)

    z = jnp.pad(z_q.reshape(Nn, code), ((0, N_PAD - Nn), (0, 0)))
    bi2 = bi.reshape(1, HID)

    h = _proj(z, Wi, bi2)
    for l in range(L):
        agg = _sc_agg(h, src, dst)
        args = (h, agg, Wc1[l], bc1[l].reshape(1, HID), Wc2[l],
                bc2[l].reshape(1, HID), gamma[l].reshape(1, HID),
                beta[l].reshape(1, HID))
        if l < L - 1:
            h = _layer(*args)
        else:
            h = _layer_out(*args, Wo1, bo1.reshape(1, HID),
                           Wo2, bo2.reshape(1, HID))
    return h[:Nn].reshape(Bb, Nn, HID)


# speedup vs baseline: 7.0138x; 2.4031x over previous
import functools

import jax
import jax.numpy as jnp
from jax import lax
from jax.experimental import pallas as pl
from jax.experimental.pallas import tpu as pltpu
from jax.experimental.pallas import tpu_sc as plsc

HID = 128
N_NODES = 10000
N_PAD = 10240
E_EDGES = 320000
GROUP = 128
N_TILES = 32
G_TOTAL = 2560
G_PER_TILE = G_TOTAL // N_TILES
E_PAD = G_TOTAL * GROUP
ROWS_PER_TILE = N_PAD // 16
DUMMY_DST = N_NODES + 16

TC_BLOCK = 512
TC_GRID = N_PAD // TC_BLOCK


def _sc_agg_body(h_hbm, srcg_hbm, dstg_hbm, out_hbm,
                 src_v, dst_v, rows_v, agg_sh, sem):
    c = lax.axis_index("c")
    s = lax.axis_index("s")
    tile = c * 16 + s

    pltpu.sync_copy(srcg_hbm.at[pl.ds(tile * G_PER_TILE, G_PER_TILE)], src_v)
    pltpu.sync_copy(dstg_hbm.at[pl.ds(tile * G_PER_TILE, G_PER_TILE)], dst_v)

    def zrow(i, carry):
        for j in range(8):
            rows_v[i, pl.ds(j * 16, 16)] = jnp.zeros((16,), jnp.float32)
        return carry
    lax.fori_loop(0, GROUP, zrow, 0)
    for z in range(ROWS_PER_TILE // GROUP):
        pltpu.sync_copy(rows_v, agg_sh.at[pl.ds(s * ROWS_PER_TILE + z * GROUP, GROUP)])
    plsc.subcore_barrier()

    def body(g, carry):
        pltpu.async_copy(h_hbm.at[src_v.at[g]], rows_v, sem).wait()
        pltpu.sync_copy(rows_v, agg_sh.at[dst_v.at[g]], add=True)
        return carry
    lax.fori_loop(0, G_PER_TILE, body, 0)
    plsc.subcore_barrier()

    for z in range(ROWS_PER_TILE // GROUP):
        r0 = s * ROWS_PER_TILE + z * GROUP
        pltpu.sync_copy(agg_sh.at[pl.ds(r0, GROUP)], rows_v)
        pltpu.sync_copy(rows_v, out_hbm.at[c, pl.ds(r0, GROUP)])


_sc_agg = pl.kernel(
    _sc_agg_body,
    out_type=jax.ShapeDtypeStruct((2, N_PAD, HID), jnp.float32),
    mesh=plsc.VectorSubcoreMesh(core_axis_name="c", subcore_axis_name="s"),
    scratch_types=[
        pltpu.VMEM((G_PER_TILE, GROUP), jnp.int32),
        pltpu.VMEM((G_PER_TILE, GROUP), jnp.int32),
        pltpu.VMEM((GROUP, HID), jnp.float32),
        pltpu.VMEM_SHARED((N_PAD, HID), jnp.float32),
        pltpu.SemaphoreType.DMA,
    ],
)


def _proj_body(z_ref, wi_ref, bi_ref, o_ref):
    o_ref[...] = (
        jnp.dot(z_ref[...], wi_ref[...], preferred_element_type=jnp.float32)
        + bi_ref[...]
    )


def _layer_body(h_ref, a_ref, w1_ref, b1_ref, w2_ref, b2_ref, g_ref, be_ref,
                o_ref):
    h = h_ref[...]
    x = h + a_ref[0] + a_ref[1]
    t = jnp.maximum(
        jnp.dot(x, w1_ref[...], preferred_element_type=jnp.float32) + b1_ref[...],
        0.0)
    t = jnp.dot(t, w2_ref[...], preferred_element_type=jnp.float32) + b2_ref[...]
    mu = jnp.mean(t, axis=-1, keepdims=True)
    var = jnp.mean((t - mu) ** 2, axis=-1, keepdims=True)
    t = (t - mu) / jnp.sqrt(var + 1e-5) * g_ref[...] + be_ref[...]
    o_ref[...] = h + jnp.maximum(t, 0.0)


def _layer_out_body(h_ref, a_ref, w1_ref, b1_ref, w2_ref, b2_ref, g_ref,
                    be_ref, wo1_ref, bo1_ref, wo2_ref, bo2_ref, o_ref):
    h = h_ref[...]
    x = h + a_ref[0] + a_ref[1]
    t = jnp.maximum(
        jnp.dot(x, w1_ref[...], preferred_element_type=jnp.float32) + b1_ref[...],
        0.0)
    t = jnp.dot(t, w2_ref[...], preferred_element_type=jnp.float32) + b2_ref[...]
    mu = jnp.mean(t, axis=-1, keepdims=True)
    var = jnp.mean((t - mu) ** 2, axis=-1, keepdims=True)
    t = (t - mu) / jnp.sqrt(var + 1e-5) * g_ref[...] + be_ref[...]
    h = h + jnp.maximum(t, 0.0)
    u = jnp.maximum(
        jnp.dot(h, wo1_ref[...], preferred_element_type=jnp.float32)
        + bo1_ref[...], 0.0)
    o_ref[...] = (
        jnp.dot(u, wo2_ref[...], preferred_element_type=jnp.float32)
        + bo2_ref[...])


def _row_spec():
    return pl.BlockSpec((TC_BLOCK, HID), lambda i: (i, 0))


def _agg_spec():
    return pl.BlockSpec((2, TC_BLOCK, HID), lambda i: (0, i, 0))


def _w_spec():
    return pl.BlockSpec((HID, HID), lambda i: (0, 0))


def _b_spec():
    return pl.BlockSpec((1, HID), lambda i: (0, 0))


_proj = pl.pallas_call(
    _proj_body,
    grid=(TC_GRID,),
    in_specs=[_row_spec(), _w_spec(), _b_spec()],
    out_specs=_row_spec(),
    out_shape=jax.ShapeDtypeStruct((N_PAD, HID), jnp.float32),
)

_layer = pl.pallas_call(
    _layer_body,
    grid=(TC_GRID,),
    in_specs=[_row_spec(), _agg_spec(), _w_spec(), _b_spec(), _w_spec(),
              _b_spec(), _b_spec(), _b_spec()],
    out_specs=_row_spec(),
    out_shape=jax.ShapeDtypeStruct((N_PAD, HID), jnp.float32),
)

_layer_out = pl.pallas_call(
    _layer_out_body,
    grid=(TC_GRID,),
    in_specs=[_row_spec(), _agg_spec(), _w_spec(), _b_spec(), _w_spec(),
              _b_spec(), _b_spec(), _b_spec(), _w_spec(), _b_spec(),
              _w_spec(), _b_spec()],
    out_specs=_row_spec(),
    out_shape=jax.ShapeDtypeStruct((N_PAD, HID), jnp.float32),
)


def kernel(z_q, edge_index, Wi, bi, Wc1, bc1, Wc2, bc2, gamma, beta,
           Wo1, bo1, Wo2, bo2):
    Bb, Nn, code = z_q.shape
    L = Wc1.shape[0]

    ei = edge_index.reshape(2, -1).astype(jnp.int32)
    pad_rows = Nn + (jnp.arange(E_PAD - E_EDGES, dtype=jnp.int32) % 128)
    src = jnp.concatenate([ei[0], pad_rows]).reshape(G_TOTAL, GROUP)
    dst = jnp.concatenate([ei[1], pad_rows]).reshape(G_TOTAL, GROUP)

    z = jnp.pad(z_q.reshape(Nn, code), ((0, N_PAD - Nn), (0, 0)))
    bi2 = bi.reshape(1, HID)

    h = _proj(z, Wi, bi2)
    for l in range(L):
        agg = _sc_agg(h, src, dst)
        args = (h, agg, Wc1[l], bc1[l].reshape(1, HID), Wc2[l],
                bc2[l].reshape(1, HID), gamma[l].reshape(1, HID),
                beta[l].reshape(1, HID))
        if l < L - 1:
            h = _layer(*args)
        else:
            h = _layer_out(*args, Wo1, bo1.reshape(1, HID),
                           Wo2, bo2.reshape(1, HID))
    return h[:Nn].reshape(Bb, Nn, HID)


# speedup vs baseline: 8.9554x; 1.2768x over previous
import functools

import jax
import jax.numpy as jnp
from jax import lax
from jax.experimental import pallas as pl
from jax.experimental.pallas import tpu as pltpu
from jax.experimental.pallas import tpu_sc as plsc

HID = 128
N_NODES = 10000
N_PAD = 10240
E_EDGES = 320000
GROUP = 128
N_TILES = 32
G_TOTAL = 2560
G_PER_TILE = G_TOTAL // N_TILES
E_PAD = G_TOTAL * GROUP
ROWS_PER_TILE = N_PAD // 16
DUMMY_DST = N_NODES + 16

TC_BLOCK = 512
TC_GRID = N_PAD // TC_BLOCK


CH = 8
NCH = G_PER_TILE // CH


def _sc_agg_body(h_hbm, srcg_hbm, dstg_hbm, out_hbm,
                 src_v, dst_v, rows_v, agg_sh, semA, semB, semIS, semID):
    c = lax.axis_index("c")
    s = lax.axis_index("s")
    tile = c * 16 + s
    base = tile * G_PER_TILE
    sems = (semA, semB)

    def zrow(i, carry):
        for j in range(8):
            rows_v[0, i, pl.ds(j * 16, 16)] = jnp.zeros((16,), jnp.float32)
        return carry
    lax.fori_loop(0, GROUP, zrow, 0)
    for z in range(ROWS_PER_TILE // GROUP):
        pltpu.sync_copy(rows_v.at[0],
                        agg_sh.at[pl.ds(s * ROWS_PER_TILE + z * GROUP, GROUP)])

    pltpu.sync_copy(srcg_hbm.at[pl.ds(base, CH)], src_v.at[0])
    pltpu.sync_copy(dstg_hbm.at[pl.ds(base, CH)], dst_v.at[0])
    pltpu.async_copy(h_hbm.at[src_v.at[0, 0]], rows_v.at[0], semA)
    plsc.subcore_barrier()

    def body(ci, carry):
        cp = lax.rem(ci, 2)
        cq = 1 - cp
        cnext = jnp.minimum(ci + 1, NCH - 1)
        pf_s = pltpu.async_copy(
            srcg_hbm.at[pl.ds(base + cnext * CH, CH)], src_v.at[cq], semIS)
        pf_d = pltpu.async_copy(
            dstg_hbm.at[pl.ds(base + cnext * CH, CH)], dst_v.at[cq], semID)
        for j in range(CH):
            p = j % 2
            q = 1 - p
            pltpu.make_async_copy(
                h_hbm.at[src_v.at[cp, j]], rows_v.at[p], sems[p]).wait()
            if j < CH - 1:
                pltpu.async_copy(
                    h_hbm.at[src_v.at[cp, j + 1]], rows_v.at[q], sems[q])
            else:
                pf_s.wait()
                pf_d.wait()
                pltpu.async_copy(
                    h_hbm.at[src_v.at[cq, 0]], rows_v.at[q], sems[q])
            pltpu.sync_copy(rows_v.at[p], agg_sh.at[dst_v.at[cp, j]], add=True)
        return carry

    lax.fori_loop(0, NCH, body, 0)
    pltpu.make_async_copy(h_hbm.at[src_v.at[0, 0]], rows_v.at[0], semA).wait()
    plsc.subcore_barrier()

    for z in range(ROWS_PER_TILE // GROUP):
        r0 = s * ROWS_PER_TILE + z * GROUP
        pltpu.sync_copy(agg_sh.at[pl.ds(r0, GROUP)], rows_v.at[0])
        pltpu.sync_copy(rows_v.at[0], out_hbm.at[c, pl.ds(r0, GROUP)])


_sc_agg = pl.kernel(
    _sc_agg_body,
    out_type=jax.ShapeDtypeStruct((2, N_PAD, HID), jnp.float32),
    mesh=plsc.VectorSubcoreMesh(core_axis_name="c", subcore_axis_name="s"),
    scratch_types=[
        pltpu.VMEM((2, CH, GROUP), jnp.int32),
        pltpu.VMEM((2, CH, GROUP), jnp.int32),
        pltpu.VMEM((2, GROUP, HID), jnp.float32),
        pltpu.VMEM_SHARED((N_PAD, HID), jnp.float32),
        pltpu.SemaphoreType.DMA,
        pltpu.SemaphoreType.DMA,
        pltpu.SemaphoreType.DMA,
        pltpu.SemaphoreType.DMA,
    ],
)


def _proj_body(z_ref, wi_ref, bi_ref, o_ref):
    o_ref[...] = (
        jnp.dot(z_ref[...], wi_ref[...], preferred_element_type=jnp.float32)
        + bi_ref[...]
    )


def _layer_body(h_ref, a_ref, w1_ref, b1_ref, w2_ref, b2_ref, g_ref, be_ref,
                o_ref):
    h = h_ref[...]
    x = h + a_ref[0] + a_ref[1]
    t = jnp.maximum(
        jnp.dot(x, w1_ref[...], preferred_element_type=jnp.float32) + b1_ref[...],
        0.0)
    t = jnp.dot(t, w2_ref[...], preferred_element_type=jnp.float32) + b2_ref[...]
    mu = jnp.mean(t, axis=-1, keepdims=True)
    var = jnp.mean((t - mu) ** 2, axis=-1, keepdims=True)
    t = (t - mu) / jnp.sqrt(var + 1e-5) * g_ref[...] + be_ref[...]
    o_ref[...] = h + jnp.maximum(t, 0.0)


def _layer_out_body(h_ref, a_ref, w1_ref, b1_ref, w2_ref, b2_ref, g_ref,
                    be_ref, wo1_ref, bo1_ref, wo2_ref, bo2_ref, o_ref):
    h = h_ref[...]
    x = h + a_ref[0] + a_ref[1]
    t = jnp.maximum(
        jnp.dot(x, w1_ref[...], preferred_element_type=jnp.float32) + b1_ref[...],
        0.0)
    t = jnp.dot(t, w2_ref[...], preferred_element_type=jnp.float32) + b2_ref[...]
    mu = jnp.mean(t, axis=-1, keepdims=True)
    var = jnp.mean((t - mu) ** 2, axis=-1, keepdims=True)
    t = (t - mu) / jnp.sqrt(var + 1e-5) * g_ref[...] + be_ref[...]
    h = h + jnp.maximum(t, 0.0)
    u = jnp.maximum(
        jnp.dot(h, wo1_ref[...], preferred_element_type=jnp.float32)
        + bo1_ref[...], 0.0)
    o_ref[...] = (
        jnp.dot(u, wo2_ref[...], preferred_element_type=jnp.float32)
        + bo2_ref[...])


def _row_spec():
    return pl.BlockSpec((TC_BLOCK, HID), lambda i: (i, 0))


def _agg_spec():
    return pl.BlockSpec((2, TC_BLOCK, HID), lambda i: (0, i, 0))


def _w_spec():
    return pl.BlockSpec((HID, HID), lambda i: (0, 0))


def _b_spec():
    return pl.BlockSpec((1, HID), lambda i: (0, 0))


_proj = pl.pallas_call(
    _proj_body,
    grid=(TC_GRID,),
    in_specs=[_row_spec(), _w_spec(), _b_spec()],
    out_specs=_row_spec(),
    out_shape=jax.ShapeDtypeStruct((N_PAD, HID), jnp.float32),
)

_layer = pl.pallas_call(
    _layer_body,
    grid=(TC_GRID,),
    in_specs=[_row_spec(), _agg_spec(), _w_spec(), _b_spec(), _w_spec(),
              _b_spec(), _b_spec(), _b_spec()],
    out_specs=_row_spec(),
    out_shape=jax.ShapeDtypeStruct((N_PAD, HID), jnp.float32),
)

_layer_out = pl.pallas_call(
    _layer_out_body,
    grid=(TC_GRID,),
    in_specs=[_row_spec(), _agg_spec(), _w_spec(), _b_spec(), _w_spec(),
              _b_spec(), _b_spec(), _b_spec(), _w_spec(), _b_spec(),
              _w_spec(), _b_spec()],
    out_specs=_row_spec(),
    out_shape=jax.ShapeDtypeStruct((N_PAD, HID), jnp.float32),
)


def kernel(z_q, edge_index, Wi, bi, Wc1, bc1, Wc2, bc2, gamma, beta,
           Wo1, bo1, Wo2, bo2):
    Bb, Nn, code = z_q.shape
    L = Wc1.shape[0]

    ei = edge_index.reshape(2, -1).astype(jnp.int32)
    pad_rows = Nn + (jnp.arange(E_PAD - E_EDGES, dtype=jnp.int32) % 128)
    src = jnp.concatenate([ei[0], pad_rows]).reshape(G_TOTAL, GROUP)
    dst = jnp.concatenate([ei[1], pad_rows]).reshape(G_TOTAL, GROUP)

    z = jnp.pad(z_q.reshape(Nn, code), ((0, N_PAD - Nn), (0, 0)))
    bi2 = bi.reshape(1, HID)

    h = _proj(z, Wi, bi2)
    for l in range(L):
        agg = _sc_agg(h, src, dst)
        args = (h, agg, Wc1[l], bc1[l].reshape(1, HID), Wc2[l],
                bc2[l].reshape(1, HID), gamma[l].reshape(1, HID),
                beta[l].reshape(1, HID))
        if l < L - 1:
            h = _layer(*args)
        else:
            h = _layer_out(*args, Wo1, bo1.reshape(1, HID),
                           Wo2, bo2.reshape(1, HID))
    return h[:Nn].reshape(Bb, Nn, HID)


# speedup vs baseline: 9.2965x; 1.0381x over previous
import functools

import jax
import jax.numpy as jnp
from jax import lax
from jax.experimental import pallas as pl
from jax.experimental.pallas import tpu as pltpu
from jax.experimental.pallas import tpu_sc as plsc

HID = 128
N_NODES = 10000
N_PAD = 10240
E_EDGES = 320000
GROUP = 128
N_TILES = 32
G_TOTAL = 2560
G_PER_TILE = G_TOTAL // N_TILES
E_PAD = G_TOTAL * GROUP
ROWS_PER_TILE = N_PAD // 16
DUMMY_DST = N_NODES + 16

TC_BLOCK = 512
TC_GRID = N_PAD // TC_BLOCK


GR = 64
G64_TOTAL = E_PAD // GR
NGT = G64_TOTAL // N_TILES
CH = 8
NCH = NGT // CH
TRASH = N_NODES + 128


def _sc_agg_body(h_hbm, srcg_hbm, dstg_hbm, out_hbm,
                 src_v, dst_v, rows_v, dum_v, agg_sh,
                 semG0, semG1, semG2, semG3, semS0, semS1, semS2, semS3,
                 semIS, semID):
    c = lax.axis_index("c")
    s = lax.axis_index("s")
    tile = c * 16 + s
    base = tile * NGT
    semG = (semG0, semG1, semG2, semG3)
    semS = (semS0, semS1, semS2, semS3)

    def slot(b):
        return rows_v.at[pl.ds(b * GR, GR)]

    def zrow(i, carry):
        for j in range(8):
            rows_v[i, pl.ds(j * 16, 16)] = jnp.zeros((16,), jnp.float32)
        return carry
    lax.fori_loop(0, 128, zrow, 0)
    for z in range(ROWS_PER_TILE // 128):
        pltpu.sync_copy(rows_v.at[pl.ds(0, 128)],
                        agg_sh.at[pl.ds(s * ROWS_PER_TILE + z * 128, 128)])

    for k in range(4):
        dum_v[0, pl.ds(k * 16, 16)] = (
            jnp.full((16,), TRASH + k * 16, jnp.int32)
            + lax.iota(jnp.int32, 16))

    pltpu.sync_copy(srcg_hbm.at[pl.ds(base, CH)], src_v.at[0])
    pltpu.sync_copy(dstg_hbm.at[pl.ds(base, CH)], dst_v.at[0])
    pltpu.async_copy(slot(2), agg_sh.at[dum_v.at[0]], semS2, add=True)
    pltpu.async_copy(slot(3), agg_sh.at[dum_v.at[0]], semS3, add=True)
    pltpu.async_copy(h_hbm.at[src_v.at[0, 0]], slot(0), semG0)
    pltpu.async_copy(h_hbm.at[src_v.at[0, 1]], slot(1), semG1)
    plsc.subcore_barrier()

    def body(ci, carry):
        cp = lax.rem(ci, 3)
        cnp = lax.rem(ci + 1, 3)
        cnext = jnp.minimum(ci + 1, NCH - 1)
        pf_s = pltpu.async_copy(
            srcg_hbm.at[pl.ds(base + cnext * CH, CH)], src_v.at[cnp], semIS)
        pf_d = pltpu.async_copy(
            dstg_hbm.at[pl.ds(base + cnext * CH, CH)], dst_v.at[cnp], semID)
        for j in range(CH):
            b = j % 4
            b2 = (j + 2) % 4
            pltpu.make_async_copy(
                h_hbm.at[src_v.at[0, 0]], slot(b), semG[b]).wait()
            pltpu.async_copy(slot(b), agg_sh.at[dst_v.at[cp, j]], semS[b],
                             add=True)
            pltpu.make_async_copy(
                slot(b2), agg_sh.at[dum_v.at[0]], semS[b2]).wait()
            if j < CH - 2:
                idx = src_v.at[cp, j + 2]
            else:
                if j == CH - 2:
                    pf_s.wait()
                    pf_d.wait()
                idx = src_v.at[cnp, j - (CH - 2)]
            pltpu.async_copy(h_hbm.at[idx], slot(b2), semG[b2])
        return carry

    lax.fori_loop(0, NCH, body, 0)
    pltpu.make_async_copy(h_hbm.at[src_v.at[0, 0]], slot(0), semG0).wait()
    pltpu.make_async_copy(h_hbm.at[src_v.at[0, 0]], slot(1), semG1).wait()
    pltpu.make_async_copy(slot(2), agg_sh.at[dum_v.at[0]], semS2).wait()
    pltpu.make_async_copy(slot(3), agg_sh.at[dum_v.at[0]], semS3).wait()
    plsc.subcore_barrier()

    for z in range(ROWS_PER_TILE // 128):
        r0 = s * ROWS_PER_TILE + z * 128
        pltpu.sync_copy(agg_sh.at[pl.ds(r0, 128)], rows_v.at[pl.ds(0, 128)])
        pltpu.sync_copy(rows_v.at[pl.ds(0, 128)], out_hbm.at[c, pl.ds(r0, 128)])


_sc_agg = pl.kernel(
    _sc_agg_body,
    out_type=jax.ShapeDtypeStruct((2, N_PAD, HID), jnp.float32),
    mesh=plsc.VectorSubcoreMesh(core_axis_name="c", subcore_axis_name="s"),
    scratch_types=[
        pltpu.VMEM((3, CH, GR), jnp.int32),
        pltpu.VMEM((3, CH, GR), jnp.int32),
        pltpu.VMEM((4 * GR, HID), jnp.float32),
        pltpu.VMEM((1, GR), jnp.int32),
        pltpu.VMEM_SHARED((N_PAD, HID), jnp.float32),
        pltpu.SemaphoreType.DMA,
        pltpu.SemaphoreType.DMA,
        pltpu.SemaphoreType.DMA,
        pltpu.SemaphoreType.DMA,
        pltpu.SemaphoreType.DMA,
        pltpu.SemaphoreType.DMA,
        pltpu.SemaphoreType.DMA,
        pltpu.SemaphoreType.DMA,
        pltpu.SemaphoreType.DMA,
        pltpu.SemaphoreType.DMA,
    ],
)


def _proj_body(z_ref, wi_ref, bi_ref, o_ref):
    o_ref[...] = (
        jnp.dot(z_ref[...], wi_ref[...], preferred_element_type=jnp.float32)
        + bi_ref[...]
    )


def _layer_body(h_ref, a_ref, w1_ref, b1_ref, w2_ref, b2_ref, g_ref, be_ref,
                o_ref):
    h = h_ref[...]
    x = h + a_ref[0] + a_ref[1]
    t = jnp.maximum(
        jnp.dot(x, w1_ref[...], preferred_element_type=jnp.float32) + b1_ref[...],
        0.0)
    t = jnp.dot(t, w2_ref[...], preferred_element_type=jnp.float32) + b2_ref[...]
    mu = jnp.mean(t, axis=-1, keepdims=True)
    var = jnp.mean((t - mu) ** 2, axis=-1, keepdims=True)
    t = (t - mu) / jnp.sqrt(var + 1e-5) * g_ref[...] + be_ref[...]
    o_ref[...] = h + jnp.maximum(t, 0.0)


def _layer_out_body(h_ref, a_ref, w1_ref, b1_ref, w2_ref, b2_ref, g_ref,
                    be_ref, wo1_ref, bo1_ref, wo2_ref, bo2_ref, o_ref):
    h = h_ref[...]
    x = h + a_ref[0] + a_ref[1]
    t = jnp.maximum(
        jnp.dot(x, w1_ref[...], preferred_element_type=jnp.float32) + b1_ref[...],
        0.0)
    t = jnp.dot(t, w2_ref[...], preferred_element_type=jnp.float32) + b2_ref[...]
    mu = jnp.mean(t, axis=-1, keepdims=True)
    var = jnp.mean((t - mu) ** 2, axis=-1, keepdims=True)
    t = (t - mu) / jnp.sqrt(var + 1e-5) * g_ref[...] + be_ref[...]
    h = h + jnp.maximum(t, 0.0)
    u = jnp.maximum(
        jnp.dot(h, wo1_ref[...], preferred_element_type=jnp.float32)
        + bo1_ref[...], 0.0)
    o_ref[...] = (
        jnp.dot(u, wo2_ref[...], preferred_element_type=jnp.float32)
        + bo2_ref[...])


def _row_spec():
    return pl.BlockSpec((TC_BLOCK, HID), lambda i: (i, 0))


def _agg_spec():
    return pl.BlockSpec((2, TC_BLOCK, HID), lambda i: (0, i, 0))


def _w_spec():
    return pl.BlockSpec((HID, HID), lambda i: (0, 0))


def _b_spec():
    return pl.BlockSpec((1, HID), lambda i: (0, 0))


_proj = pl.pallas_call(
    _proj_body,
    grid=(TC_GRID,),
    in_specs=[_row_spec(), _w_spec(), _b_spec()],
    out_specs=_row_spec(),
    out_shape=jax.ShapeDtypeStruct((N_PAD, HID), jnp.float32),
)

_layer = pl.pallas_call(
    _layer_body,
    grid=(TC_GRID,),
    in_specs=[_row_spec(), _agg_spec(), _w_spec(), _b_spec(), _w_spec(),
              _b_spec(), _b_spec(), _b_spec()],
    out_specs=_row_spec(),
    out_shape=jax.ShapeDtypeStruct((N_PAD, HID), jnp.float32),
)

_layer_out = pl.pallas_call(
    _layer_out_body,
    grid=(TC_GRID,),
    in_specs=[_row_spec(), _agg_spec(), _w_spec(), _b_spec(), _w_spec(),
              _b_spec(), _b_spec(), _b_spec(), _w_spec(), _b_spec(),
              _w_spec(), _b_spec()],
    out_specs=_row_spec(),
    out_shape=jax.ShapeDtypeStruct((N_PAD, HID), jnp.float32),
)


def kernel(z_q, edge_index, Wi, bi, Wc1, bc1, Wc2, bc2, gamma, beta,
           Wo1, bo1, Wo2, bo2):
    Bb, Nn, code = z_q.shape
    L = Wc1.shape[0]

    ei = edge_index.reshape(2, -1).astype(jnp.int32)
    pad_rows = Nn + (jnp.arange(E_PAD - E_EDGES, dtype=jnp.int32) % 128)
    src = jnp.concatenate([ei[0], pad_rows]).reshape(G64_TOTAL, GR)
    dst = jnp.concatenate([ei[1], pad_rows]).reshape(G64_TOTAL, GR)

    z = jnp.pad(z_q.reshape(Nn, code), ((0, N_PAD - Nn), (0, 0)))
    bi2 = bi.reshape(1, HID)

    h = _proj(z, Wi, bi2)
    for l in range(L):
        agg = _sc_agg(h, src, dst)
        args = (h, agg, Wc1[l], bc1[l].reshape(1, HID), Wc2[l],
                bc2[l].reshape(1, HID), gamma[l].reshape(1, HID),
                beta[l].reshape(1, HID))
        if l < L - 1:
            h = _layer(*args)
        else:
            h = _layer_out(*args, Wo1, bo1.reshape(1, HID),
                           Wo2, bo2.reshape(1, HID))
    return h[:Nn].reshape(Bb, Nn, HID)


# speedup vs baseline: 9.3647x; 1.0073x over previous
import functools

import jax
import jax.numpy as jnp
from jax import lax
from jax.experimental import pallas as pl
from jax.experimental.pallas import tpu as pltpu
from jax.experimental.pallas import tpu_sc as plsc

HID = 128
N_NODES = 10000
N_PAD = 10240
E_EDGES = 320000
GROUP = 128
N_TILES = 32
G_TOTAL = 2560
G_PER_TILE = G_TOTAL // N_TILES
E_PAD = G_TOTAL * GROUP
ROWS_PER_TILE = N_PAD // 16
DUMMY_DST = N_NODES + 16

TC_BLOCK = 512
TC_GRID = N_PAD // TC_BLOCK


GR = 64
G64_TOTAL = E_PAD // GR
NGT = G64_TOTAL // N_TILES
CH = 8
NCH = NGT // CH
TRASH = N_NODES + 128


def _sc_agg_body(h_hbm, srcg_hbm, dstg_hbm, out_hbm,
                 src_v, dst_v, rows_v, dum_v, agg_sh,
                 semG0, semG1, semG2, semG3, semS0, semS1, semS2, semS3,
                 semIS, semID):
    c = lax.axis_index("c")
    s = lax.axis_index("s")
    tile = c * 16 + s
    base = tile * NGT
    semG = (semG0, semG1, semG2, semG3)
    semS = (semS0, semS1, semS2, semS3)

    def slot(b):
        return rows_v.at[pl.ds(b * GR, GR)]

    def zrow(i, carry):
        for j in range(8):
            rows_v[i, pl.ds(j * 16, 16)] = jnp.zeros((16,), jnp.float32)
        return carry
    lax.fori_loop(0, 128, zrow, 0)
    for z in range(ROWS_PER_TILE // 128):
        pltpu.async_copy(rows_v.at[pl.ds(0, 128)],
                         agg_sh.at[pl.ds(s * ROWS_PER_TILE + z * 128, 128)],
                         semIS)
    for z in range(ROWS_PER_TILE // 128):
        pltpu.make_async_copy(
            rows_v.at[pl.ds(0, 128)],
            agg_sh.at[pl.ds(s * ROWS_PER_TILE, 128)], semIS).wait()

    for k in range(4):
        dum_v[0, pl.ds(k * 16, 16)] = (
            jnp.full((16,), TRASH + k * 16, jnp.int32)
            + lax.iota(jnp.int32, 16))

    pltpu.sync_copy(srcg_hbm.at[pl.ds(base, CH)], src_v.at[0])
    pltpu.sync_copy(dstg_hbm.at[pl.ds(base, CH)], dst_v.at[0])
    pltpu.async_copy(slot(2), agg_sh.at[dum_v.at[0]], semS2, add=True)
    pltpu.async_copy(slot(3), agg_sh.at[dum_v.at[0]], semS3, add=True)
    pltpu.async_copy(h_hbm.at[src_v.at[0, 0]], slot(0), semG0)
    pltpu.async_copy(h_hbm.at[src_v.at[0, 1]], slot(1), semG1)
    plsc.subcore_barrier()

    def body(ci, carry):
        cp = lax.rem(ci, 3)
        cnp = lax.rem(ci + 1, 3)
        cnext = jnp.minimum(ci + 1, NCH - 1)
        pf_s = pltpu.async_copy(
            srcg_hbm.at[pl.ds(base + cnext * CH, CH)], src_v.at[cnp], semIS)
        pf_d = pltpu.async_copy(
            dstg_hbm.at[pl.ds(base + cnext * CH, CH)], dst_v.at[cnp], semID)
        for j in range(CH):
            b = j % 4
            b2 = (j + 2) % 4
            pltpu.make_async_copy(
                h_hbm.at[src_v.at[0, 0]], slot(b), semG[b]).wait()
            pltpu.async_copy(slot(b), agg_sh.at[dst_v.at[cp, j]], semS[b],
                             add=True)
            pltpu.make_async_copy(
                slot(b2), agg_sh.at[dum_v.at[0]], semS[b2]).wait()
            if j < CH - 2:
                idx = src_v.at[cp, j + 2]
            else:
                if j == CH - 2:
                    pf_s.wait()
                    pf_d.wait()
                idx = src_v.at[cnp, j - (CH - 2)]
            pltpu.async_copy(h_hbm.at[idx], slot(b2), semG[b2])
        return carry

    lax.fori_loop(0, NCH, body, 0)
    pltpu.make_async_copy(h_hbm.at[src_v.at[0, 0]], slot(0), semG0).wait()
    pltpu.make_async_copy(h_hbm.at[src_v.at[0, 0]], slot(1), semG1).wait()
    pltpu.make_async_copy(slot(2), agg_sh.at[dum_v.at[0]], semS2).wait()
    pltpu.make_async_copy(slot(3), agg_sh.at[dum_v.at[0]], semS3).wait()
    plsc.subcore_barrier()

    nwb = ROWS_PER_TILE // 128
    for z in range(nwb):
        blk = (z % 2) * 128
        r0 = s * ROWS_PER_TILE + z * 128
        if z >= 2:
            pltpu.make_async_copy(
                rows_v.at[pl.ds(blk, 128)],
                out_hbm.at[c, pl.ds(s * ROWS_PER_TILE, 128)], semID).wait()
        pltpu.sync_copy(agg_sh.at[pl.ds(r0, 128)], rows_v.at[pl.ds(blk, 128)])
        pltpu.async_copy(rows_v.at[pl.ds(blk, 128)],
                         out_hbm.at[c, pl.ds(r0, 128)], semID)
    for z in range(2):
        pltpu.make_async_copy(
            rows_v.at[pl.ds(0, 128)],
            out_hbm.at[c, pl.ds(s * ROWS_PER_TILE, 128)], semID).wait()


_sc_agg = pl.kernel(
    _sc_agg_body,
    out_type=jax.ShapeDtypeStruct((2, N_PAD, HID), jnp.float32),
    mesh=plsc.VectorSubcoreMesh(core_axis_name="c", subcore_axis_name="s"),
    scratch_types=[
        pltpu.VMEM((3, CH, GR), jnp.int32),
        pltpu.VMEM((3, CH, GR), jnp.int32),
        pltpu.VMEM((4 * GR, HID), jnp.float32),
        pltpu.VMEM((1, GR), jnp.int32),
        pltpu.VMEM_SHARED((N_PAD, HID), jnp.float32),
        pltpu.SemaphoreType.DMA,
        pltpu.SemaphoreType.DMA,
        pltpu.SemaphoreType.DMA,
        pltpu.SemaphoreType.DMA,
        pltpu.SemaphoreType.DMA,
        pltpu.SemaphoreType.DMA,
        pltpu.SemaphoreType.DMA,
        pltpu.SemaphoreType.DMA,
        pltpu.SemaphoreType.DMA,
        pltpu.SemaphoreType.DMA,
    ],
)


def _proj_body(z_ref, wi_ref, bi_ref, o_ref):
    o_ref[...] = (
        jnp.dot(z_ref[...], wi_ref[...], preferred_element_type=jnp.float32)
        + bi_ref[...]
    )


def _layer_body(h_ref, a_ref, w1_ref, b1_ref, w2_ref, b2_ref, g_ref, be_ref,
                o_ref):
    h = h_ref[...]
    x = h + a_ref[0] + a_ref[1]
    t = jnp.maximum(
        jnp.dot(x, w1_ref[...], preferred_element_type=jnp.float32) + b1_ref[...],
        0.0)
    t = jnp.dot(t, w2_ref[...], preferred_element_type=jnp.float32) + b2_ref[...]
    mu = jnp.mean(t, axis=-1, keepdims=True)
    var = jnp.mean((t - mu) ** 2, axis=-1, keepdims=True)
    t = (t - mu) / jnp.sqrt(var + 1e-5) * g_ref[...] + be_ref[...]
    o_ref[...] = h + jnp.maximum(t, 0.0)


def _layer_out_body(h_ref, a_ref, w1_ref, b1_ref, w2_ref, b2_ref, g_ref,
                    be_ref, wo1_ref, bo1_ref, wo2_ref, bo2_ref, o_ref):
    h = h_ref[...]
    x = h + a_ref[0] + a_ref[1]
    t = jnp.maximum(
        jnp.dot(x, w1_ref[...], preferred_element_type=jnp.float32) + b1_ref[...],
        0.0)
    t = jnp.dot(t, w2_ref[...], preferred_element_type=jnp.float32) + b2_ref[...]
    mu = jnp.mean(t, axis=-1, keepdims=True)
    var = jnp.mean((t - mu) ** 2, axis=-1, keepdims=True)
    t = (t - mu) / jnp.sqrt(var + 1e-5) * g_ref[...] + be_ref[...]
    h = h + jnp.maximum(t, 0.0)
    u = jnp.maximum(
        jnp.dot(h, wo1_ref[...], preferred_element_type=jnp.float32)
        + bo1_ref[...], 0.0)
    o_ref[...] = (
        jnp.dot(u, wo2_ref[...], preferred_element_type=jnp.float32)
        + bo2_ref[...])


def _row_spec():
    return pl.BlockSpec((TC_BLOCK, HID), lambda i: (i, 0))


def _agg_spec():
    return pl.BlockSpec((2, TC_BLOCK, HID), lambda i: (0, i, 0))


def _w_spec():
    return pl.BlockSpec((HID, HID), lambda i: (0, 0))


def _b_spec():
    return pl.BlockSpec((1, HID), lambda i: (0, 0))


_proj = pl.pallas_call(
    _proj_body,
    grid=(TC_GRID,),
    in_specs=[_row_spec(), _w_spec(), _b_spec()],
    out_specs=_row_spec(),
    out_shape=jax.ShapeDtypeStruct((N_PAD, HID), jnp.float32),
)

_layer = pl.pallas_call(
    _layer_body,
    grid=(TC_GRID,),
    in_specs=[_row_spec(), _agg_spec(), _w_spec(), _b_spec(), _w_spec(),
              _b_spec(), _b_spec(), _b_spec()],
    out_specs=_row_spec(),
    out_shape=jax.ShapeDtypeStruct((N_PAD, HID), jnp.float32),
)

_layer_out = pl.pallas_call(
    _layer_out_body,
    grid=(TC_GRID,),
    in_specs=[_row_spec(), _agg_spec(), _w_spec(), _b_spec(), _w_spec(),
              _b_spec(), _b_spec(), _b_spec(), _w_spec(), _b_spec(),
              _w_spec(), _b_spec()],
    out_specs=_row_spec(),
    out_shape=jax.ShapeDtypeStruct((N_PAD, HID), jnp.float32),
)


def kernel(z_q, edge_index, Wi, bi, Wc1, bc1, Wc2, bc2, gamma, beta,
           Wo1, bo1, Wo2, bo2):
    Bb, Nn, code = z_q.shape
    L = Wc1.shape[0]

    ei = edge_index.reshape(2, -1).astype(jnp.int32)
    pad_rows = Nn + (jnp.arange(E_PAD - E_EDGES, dtype=jnp.int32) % 128)
    src = jnp.concatenate([ei[0], pad_rows]).reshape(G64_TOTAL, GR)
    dst = jnp.concatenate([ei[1], pad_rows]).reshape(G64_TOTAL, GR)

    z = jnp.pad(z_q.reshape(Nn, code), ((0, N_PAD - Nn), (0, 0)))
    bi2 = bi.reshape(1, HID)

    h = _proj(z, Wi, bi2)
    for l in range(L):
        agg = _sc_agg(h, src, dst)
        args = (h, agg, Wc1[l], bc1[l].reshape(1, HID), Wc2[l],
                bc2[l].reshape(1, HID), gamma[l].reshape(1, HID),
                beta[l].reshape(1, HID))
        if l < L - 1:
            h = _layer(*args)
        else:
            h = _layer_out(*args, Wo1, bo1.reshape(1, HID),
                           Wo2, bo2.reshape(1, HID))
    return h[:Nn].reshape(Bb, Nn, HID)
